# baseline (speedup 1.0000x reference)
import dataclasses
import functools

import jax
import jax.numpy as jnp
from jax import lax
from jax.experimental import pallas as pl
from jax.experimental.pallas import tpu as pltpu
from jax.experimental.pallas import tpu_sc as plsc

HID = 768
INTER = 2048
NEXP = 64
N_TOK = 8192
CB = 512
NCB = N_TOK // CB
TM = 128
NT = N_TOK // TM + NEXP - 1
NW = 32
TOK_W = N_TOK // NW
CHUNK = 128
NCH = TOK_W // CHUNK


def _count_body(e_ref, rank_ref, counts_ref, carry):
    i = pl.program_id(0)

    @pl.when(i == 0)
    def _():
        carry[...] = jnp.zeros_like(carry)

    e = e_ref[0, 0, :]
    k = lax.broadcasted_iota(jnp.int32, (CB, NEXP), 1)
    onehot = (k == e[:, None]).astype(jnp.float32)
    a = lax.broadcasted_iota(jnp.int32, (CB, CB), 0)
    b = lax.broadcasted_iota(jnp.int32, (CB, CB), 1)
    tri = (a > b).astype(jnp.float32)
    cum = jnp.dot(tri, onehot, preferred_element_type=jnp.float32)
    cum = cum + carry[...]
    rank_ref[0, 0, :] = jnp.sum(onehot * cum, axis=1).astype(jnp.int32)
    carry[...] = carry[...] + jnp.sum(onehot, axis=0, keepdims=True)
    counts_ref[...] = carry[...]


def _count_call(e3):
    return pl.pallas_call(
        _count_body,
        grid=(NCB,),
        in_specs=[pl.BlockSpec((1, 1, CB), lambda i: (i, 0, 0))],
        out_specs=[
            pl.BlockSpec((1, 1, CB), lambda i: (i, 0, 0)),
            pl.BlockSpec((1, NEXP), lambda i: (0, 0)),
        ],
        out_shape=[
            jax.ShapeDtypeStruct((NCB, 1, CB), jnp.int32),
            jax.ShapeDtypeStruct((1, NEXP), jnp.float32),
        ],
        scratch_shapes=[pltpu.VMEM((1, NEXP), jnp.float32)],
    )(e3)


def _sc_compiler_params():
    cp = pltpu.CompilerParams()
    if "needs_layout_passes" in pltpu.CompilerParams.__dataclass_fields__:
        cp = dataclasses.replace(cp, needs_layout_passes=False)
    return cp


def _dispatch_call(x2, e, r, offs):
    mesh = plsc.VectorSubcoreMesh(core_axis_name="c", subcore_axis_name="s")

    @functools.partial(
        pl.kernel,
        compiler_params=_sc_compiler_params(),
        out_type=[
            jax.ShapeDtypeStruct((N_TOK, HID), jnp.float32),
            jax.ShapeDtypeStruct((NW * NCH, CHUNK), jnp.int32),
        ],
        mesh=mesh,
        scratch_types=[
            pltpu.VMEM((NEXP,), jnp.int32),
            pltpu.VMEM((CHUNK,), jnp.int32),
            pltpu.VMEM((CHUNK,), jnp.int32),
            pltpu.VMEM((CHUNK,), jnp.int32),
            pltpu.VMEM((CHUNK, HID), jnp.float32),
        ],
    )
    def k2(x_hbm, e_hbm, r_hbm, off_hbm, xs_hbm, dest_hbm,
           off_vm, e_vm, r_vm, d_vm, rows_vm):
        wid = lax.axis_index("c") * 16 + lax.axis_index("s")
        pltpu.sync_copy(off_hbm, off_vm)
        for c in range(NCH):
            base = wid * TOK_W + c * CHUNK
            pltpu.sync_copy(e_hbm.at[pl.ds(base, CHUNK)], e_vm)
            pltpu.sync_copy(r_hbm.at[pl.ds(base, CHUNK)], r_vm)

            @pl.loop(0, CHUNK // 16)
            def _(j):
                ev = e_vm[pl.ds(j * 16, 16)]
                rv = r_vm[pl.ds(j * 16, 16)]
                ov = plsc.load_gather(off_vm, [ev])
                d_vm[pl.ds(j * 16, 16)] = ov + rv

            pltpu.sync_copy(d_vm, dest_hbm.at[wid * NCH + c])
            pltpu.sync_copy(x_hbm.at[pl.ds(base, CHUNK)], rows_vm)
            pltpu.sync_copy(rows_vm, xs_hbm.at[d_vm])

    return k2(x2, e, r, offs)


def _ffn_body(meta_ref, xs_ref, We1_ref, be1_ref, We2_ref, be2_ref,
              Ws1_ref, bs1_ref, Ws2_ref, bs2_ref, out_ref):
    s = pl.program_id(0)

    @pl.when(meta_ref[2, s] == 1)
    def _():
        xb = xs_ref[...].astype(jnp.bfloat16)
        h = jnp.dot(xb, Ws1_ref[...].astype(jnp.bfloat16),
                    preferred_element_type=jnp.float32) + bs1_ref[...]
        h = h * jax.nn.sigmoid(h)
        out_ref[...] = jnp.dot(h.astype(jnp.bfloat16),
                               Ws2_ref[...].astype(jnp.bfloat16),
                               preferred_element_type=jnp.float32) + bs2_ref[...]

    @pl.when(meta_ref[3, s] == 1)
    def _():
        xb = xs_ref[...].astype(jnp.bfloat16)
        h = jnp.dot(xb, We1_ref[0].astype(jnp.bfloat16),
                    preferred_element_type=jnp.float32) + be1_ref[0]
        h = h * jax.nn.sigmoid(h)
        y = jnp.dot(h.astype(jnp.bfloat16), We2_ref[0].astype(jnp.bfloat16),
                    preferred_element_type=jnp.float32) + be2_ref[0]
        row = meta_ref[1, s] * TM + lax.broadcasted_iota(jnp.int32, (TM, 1), 0)
        m = (row >= meta_ref[4, s]) & (row < meta_ref[5, s])
        out_ref[...] += jnp.where(m, y, 0.0)


def _ffn_call(meta, xs, We1, be1, We2, be2, Ws1, bs1, Ws2, bs2):
    def outer(meta_ref, xs_hbm, We1_hbm, We2_hbm, be1_v, be2_v,
              Ws1_v, bs1_v, Ws2_v, bs2_v, out_hbm):

        def inner(idxs, xs_ref, We1_ref, We2_ref, out_ref):
            s = idxs[0]

            @pl.when(meta_ref[2, s] == 1)
            def _():
                xb = xs_ref[...].astype(jnp.bfloat16)
                h = jnp.dot(xb, Ws1_v[...].astype(jnp.bfloat16),
                            preferred_element_type=jnp.float32) + bs1_v[...]
                h = h * jax.nn.sigmoid(h)
                out_ref[...] = jnp.dot(
                    h.astype(jnp.bfloat16), Ws2_v[...].astype(jnp.bfloat16),
                    preferred_element_type=jnp.float32) + bs2_v[...]

            @pl.when(meta_ref[3, s] == 1)
            def _():
                eidx = meta_ref[0, s]
                xb = xs_ref[...].astype(jnp.bfloat16)
                h = jnp.dot(xb, We1_ref[0].astype(jnp.bfloat16),
                            preferred_element_type=jnp.float32)
                h = h + be1_v[pl.ds(eidx, 1)]
                h = h * jax.nn.sigmoid(h)
                y = jnp.dot(h.astype(jnp.bfloat16),
                            We2_ref[0].astype(jnp.bfloat16),
                            preferred_element_type=jnp.float32)
                y = y + be2_v[pl.ds(eidx, 1)]
                row = (meta_ref[1, s] * TM
                       + lax.broadcasted_iota(jnp.int32, (TM, 1), 0))
                m = (row >= meta_ref[4, s]) & (row < meta_ref[5, s])
                out_ref[...] += jnp.where(m, y, 0.0)

        pltpu.emit_pipeline(
            inner,
            grid=(NT,),
            in_specs=[
                pl.BlockSpec((TM, HID), lambda s: (meta_ref[1, s], 0)),
                pl.BlockSpec((1, HID, INTER), lambda s: (meta_ref[0, s], 0, 0),
                             pipeline_mode=pl.Buffered(buffer_count=3,
                                                       use_lookahead=True)),
                pl.BlockSpec((1, INTER, HID), lambda s: (meta_ref[0, s], 0, 0),
                             pipeline_mode=pl.Buffered(buffer_count=3,
                                                       use_lookahead=True)),
            ],
            out_specs=[pl.BlockSpec((TM, HID), lambda s: (meta_ref[1, s], 0))],
            _explicit_indices=True,
        )(xs_hbm, We1_hbm, We2_hbm, out_hbm)

    grid_spec = pltpu.PrefetchScalarGridSpec(
        num_scalar_prefetch=1,
        grid=(1,),
        in_specs=[
            pl.BlockSpec(memory_space=pltpu.MemorySpace.HBM),
            pl.BlockSpec(memory_space=pltpu.MemorySpace.HBM),
            pl.BlockSpec(memory_space=pltpu.MemorySpace.HBM),
            pl.BlockSpec(memory_space=pltpu.MemorySpace.VMEM),
            pl.BlockSpec(memory_space=pltpu.MemorySpace.VMEM),
            pl.BlockSpec(memory_space=pltpu.MemorySpace.VMEM),
            pl.BlockSpec(memory_space=pltpu.MemorySpace.VMEM),
            pl.BlockSpec(memory_space=pltpu.MemorySpace.VMEM),
            pl.BlockSpec(memory_space=pltpu.MemorySpace.VMEM),
        ],
        out_specs=pl.BlockSpec(memory_space=pltpu.MemorySpace.HBM),
    )
    return pl.pallas_call(
        outer,
        grid_spec=grid_spec,
        out_shape=jax.ShapeDtypeStruct((N_TOK, HID), jnp.float32),
    )(meta, xs, We1, We2, be1, be2, Ws1, bs1, Ws2, bs2)


def _combine_call(outs, dest):
    mesh = plsc.VectorSubcoreMesh(core_axis_name="c", subcore_axis_name="s")

    @functools.partial(
        pl.kernel,
        out_type=jax.ShapeDtypeStruct((N_TOK, HID), jnp.float32),
        mesh=mesh,
        scratch_types=[
            pltpu.VMEM((CHUNK,), jnp.int32),
            pltpu.VMEM((CHUNK, HID), jnp.float32),
        ],
    )
    def k4(outs_hbm, dest_hbm, o_hbm, idx_vm, rows_vm):
        wid = lax.axis_index("c") * 16 + lax.axis_index("s")
        for c in range(NCH):
            pltpu.sync_copy(dest_hbm.at[wid * NCH + c], idx_vm)
            pltpu.sync_copy(outs_hbm.at[idx_vm], rows_vm)
            pltpu.sync_copy(rows_vm, o_hbm.at[pl.ds(wid * TOK_W + c * CHUNK,
                                                    CHUNK)])

    return k4(outs, dest)


def _metadata(counts, starts, ends):
    t_first = starts // TM
    t_last = jnp.maximum(t_first, (ends - 1) // TM)
    nsteps = jnp.where(counts > 0, t_last - t_first + 1, 0)
    total = jnp.sum(nsteps)
    sexp = jnp.repeat(jnp.arange(NEXP, dtype=jnp.int32), nsteps,
                      total_repeat_length=NT)
    base = jnp.cumsum(nsteps) - nsteps
    steps = jnp.arange(NT, dtype=jnp.int32)
    stile = t_first[sexp] + steps - base[sexp]
    valid = steps < total
    last = total - 1
    sexp = jnp.where(valid, sexp, jnp.take(sexp, last))
    stile = jnp.where(valid, stile, jnp.take(stile, last))
    stile = jnp.clip(stile, 0, N_TOK // TM - 1)
    sfirst = valid & (stile != jnp.roll(stile, 1))
    sfirst = sfirst.at[0].set(True)
    meta = jnp.stack([
        sexp,
        stile,
        sfirst.astype(jnp.int32),
        valid.astype(jnp.int32),
        starts[sexp],
        ends[sexp],
    ])
    return meta.astype(jnp.int32)


def kernel(x, Wr1, br1, Wr2, br2, We1, be1, We2, be2, Ws1, bs1, Ws2, bs2):
    B, T, C = x.shape
    x2 = x.reshape(N_TOK, C)

    logits = jax.nn.relu(x2 @ Wr1 + br1) @ Wr2 + br2
    gates = jax.nn.softmax(logits, axis=-1)
    _, topk_idx = jax.lax.top_k(gates, 1)
    e = topk_idx[:, 0].astype(jnp.int32)

    rank3, counts_f = _count_call(e.reshape(NCB, 1, CB))
    counts = counts_f[0].astype(jnp.int32)
    ends = jnp.cumsum(counts)
    starts = ends - counts

    xs, dest = _dispatch_call(x2, e, rank3.reshape(N_TOK), starts)

    meta = _metadata(counts, starts, ends)
    outs = _ffn_call(meta, xs, We1, be1, We2, be2,
                     Ws1, bs1.reshape(1, INTER), Ws2, bs2.reshape(1, HID))

    out = _combine_call(outs, dest)
    return out.reshape(B, T, C)


# speedup vs baseline: 1.0519x; 1.0519x over previous
import dataclasses
import functools

import jax
import jax.numpy as jnp
from jax import lax
from jax.experimental import pallas as pl
from jax.experimental.pallas import tpu as pltpu
from jax.experimental.pallas import tpu_sc as plsc

HID = 768
INTER = 2048
NEXP = 64
N_TOK = 8192
CB = 512
NCB = N_TOK // CB
TM = 256
NT = N_TOK // TM + NEXP - 1
NW = 32
TOK_W = N_TOK // NW
CHUNK = 128
NCH = TOK_W // CHUNK


def _count_body(e_ref, rank_ref, counts_ref, carry):
    i = pl.program_id(0)

    @pl.when(i == 0)
    def _():
        carry[...] = jnp.zeros_like(carry)

    e = e_ref[0, 0, :]
    k = lax.broadcasted_iota(jnp.int32, (CB, NEXP), 1)
    onehot = (k == e[:, None]).astype(jnp.float32)
    a = lax.broadcasted_iota(jnp.int32, (CB, CB), 0)
    b = lax.broadcasted_iota(jnp.int32, (CB, CB), 1)
    tri = (a > b).astype(jnp.float32)
    cum = jnp.dot(tri, onehot, preferred_element_type=jnp.float32)
    cum = cum + carry[...]
    rank_ref[0, 0, :] = jnp.sum(onehot * cum, axis=1).astype(jnp.int32)
    carry[...] = carry[...] + jnp.sum(onehot, axis=0, keepdims=True)
    counts_ref[...] = carry[...]


def _count_call(e3):
    return pl.pallas_call(
        _count_body,
        grid=(NCB,),
        in_specs=[pl.BlockSpec((1, 1, CB), lambda i: (i, 0, 0))],
        out_specs=[
            pl.BlockSpec((1, 1, CB), lambda i: (i, 0, 0)),
            pl.BlockSpec((1, NEXP), lambda i: (0, 0)),
        ],
        out_shape=[
            jax.ShapeDtypeStruct((NCB, 1, CB), jnp.int32),
            jax.ShapeDtypeStruct((1, NEXP), jnp.float32),
        ],
        scratch_shapes=[pltpu.VMEM((1, NEXP), jnp.float32)],
    )(e3)


def _sc_compiler_params():
    cp = pltpu.CompilerParams()
    if "needs_layout_passes" in pltpu.CompilerParams.__dataclass_fields__:
        cp = dataclasses.replace(cp, needs_layout_passes=False)
    return cp


def _dispatch_call(x2, e, r, offs):
    mesh = plsc.VectorSubcoreMesh(core_axis_name="c", subcore_axis_name="s")

    @functools.partial(
        pl.kernel,
        compiler_params=_sc_compiler_params(),
        out_type=[
            jax.ShapeDtypeStruct((N_TOK, HID), jnp.float32),
            jax.ShapeDtypeStruct((NW * NCH, CHUNK), jnp.int32),
        ],
        mesh=mesh,
        scratch_types=[
            pltpu.VMEM((NEXP,), jnp.int32),
            pltpu.VMEM((CHUNK,), jnp.int32),
            pltpu.VMEM((CHUNK,), jnp.int32),
            pltpu.VMEM((CHUNK,), jnp.int32),
            pltpu.VMEM((CHUNK, HID), jnp.float32),
        ],
    )
    def k2(x_hbm, e_hbm, r_hbm, off_hbm, xs_hbm, dest_hbm,
           off_vm, e_vm, r_vm, d_vm, rows_vm):
        wid = lax.axis_index("c") * 16 + lax.axis_index("s")
        pltpu.sync_copy(off_hbm, off_vm)
        for c in range(NCH):
            base = wid * TOK_W + c * CHUNK
            pltpu.sync_copy(e_hbm.at[pl.ds(base, CHUNK)], e_vm)
            pltpu.sync_copy(r_hbm.at[pl.ds(base, CHUNK)], r_vm)

            @pl.loop(0, CHUNK // 16)
            def _(j):
                ev = e_vm[pl.ds(j * 16, 16)]
                rv = r_vm[pl.ds(j * 16, 16)]
                ov = plsc.load_gather(off_vm, [ev])
                d_vm[pl.ds(j * 16, 16)] = ov + rv

            pltpu.sync_copy(d_vm, dest_hbm.at[wid * NCH + c])
            pltpu.sync_copy(x_hbm.at[pl.ds(base, CHUNK)], rows_vm)
            pltpu.sync_copy(rows_vm, xs_hbm.at[d_vm])

    return k2(x2, e, r, offs)


def _ffn_call(meta, xs, We1, be1, We2, be2, Ws1, bs1, Ws2, bs2):
    def outer(meta_ref, xs_hbm, We1_hbm, We2_hbm, be1_v, be2_v,
              Ws1_v, bs1_v, Ws2_v, bs2_v, out_hbm):

        def inner(idxs, xs_ref, We1_ref, We2_ref, out_ref):
            s = idxs[0]

            @pl.when(meta_ref[2, s] == 1)
            def _():
                xb = xs_ref[...].astype(jnp.bfloat16)
                h = jnp.dot(xb, Ws1_v[...].astype(jnp.bfloat16),
                            preferred_element_type=jnp.float32) + bs1_v[...]
                h = h * jax.nn.sigmoid(h)
                out_ref[...] = jnp.dot(
                    h.astype(jnp.bfloat16), Ws2_v[...].astype(jnp.bfloat16),
                    preferred_element_type=jnp.float32) + bs2_v[...]

            @pl.when(meta_ref[3, s] == 1)
            def _():
                eidx = meta_ref[0, s]
                xb = xs_ref[...].astype(jnp.bfloat16)
                h = jnp.dot(xb, We1_ref[0].astype(jnp.bfloat16),
                            preferred_element_type=jnp.float32)
                h = h + be1_v[pl.ds(eidx, 1)]
                h = h * jax.nn.sigmoid(h)
                y = jnp.dot(h.astype(jnp.bfloat16),
                            We2_ref[0].astype(jnp.bfloat16),
                            preferred_element_type=jnp.float32)
                y = y + be2_v[pl.ds(eidx, 1)]
                row = (meta_ref[1, s] * TM
                       + lax.broadcasted_iota(jnp.int32, (TM, 1), 0))
                m = (row >= meta_ref[4, s]) & (row < meta_ref[5, s])
                out_ref[...] += jnp.where(m, y, 0.0)

        pltpu.emit_pipeline(
            inner,
            grid=(NT,),
            in_specs=[
                pl.BlockSpec((TM, HID), lambda s: (meta_ref[1, s], 0)),
                pl.BlockSpec((1, HID, INTER), lambda s: (meta_ref[0, s], 0, 0),
                             pipeline_mode=pl.Buffered(buffer_count=3,
                                                       use_lookahead=True)),
                pl.BlockSpec((1, INTER, HID), lambda s: (meta_ref[0, s], 0, 0),
                             pipeline_mode=pl.Buffered(buffer_count=3,
                                                       use_lookahead=True)),
            ],
            out_specs=[pl.BlockSpec((TM, HID), lambda s: (meta_ref[1, s], 0))],
            _explicit_indices=True,
        )(xs_hbm, We1_hbm, We2_hbm, out_hbm)

    grid_spec = pltpu.PrefetchScalarGridSpec(
        num_scalar_prefetch=1,
        grid=(1,),
        in_specs=[
            pl.BlockSpec(memory_space=pltpu.MemorySpace.HBM),
            pl.BlockSpec(memory_space=pltpu.MemorySpace.HBM),
            pl.BlockSpec(memory_space=pltpu.MemorySpace.HBM),
            pl.BlockSpec(memory_space=pltpu.MemorySpace.VMEM),
            pl.BlockSpec(memory_space=pltpu.MemorySpace.VMEM),
            pl.BlockSpec(memory_space=pltpu.MemorySpace.VMEM),
            pl.BlockSpec(memory_space=pltpu.MemorySpace.VMEM),
            pl.BlockSpec(memory_space=pltpu.MemorySpace.VMEM),
            pl.BlockSpec(memory_space=pltpu.MemorySpace.VMEM),
        ],
        out_specs=pl.BlockSpec(memory_space=pltpu.MemorySpace.HBM),
    )
    return pl.pallas_call(
        outer,
        grid_spec=grid_spec,
        out_shape=jax.ShapeDtypeStruct((N_TOK, HID), jnp.float32),
    )(meta, xs, We1, We2, be1, be2, Ws1, bs1, Ws2, bs2)


def _combine_call(outs, dest):
    mesh = plsc.VectorSubcoreMesh(core_axis_name="c", subcore_axis_name="s")

    @functools.partial(
        pl.kernel,
        out_type=jax.ShapeDtypeStruct((N_TOK, HID), jnp.float32),
        mesh=mesh,
        scratch_types=[
            pltpu.VMEM((CHUNK,), jnp.int32),
            pltpu.VMEM((CHUNK, HID), jnp.float32),
        ],
    )
    def k4(outs_hbm, dest_hbm, o_hbm, idx_vm, rows_vm):
        wid = lax.axis_index("c") * 16 + lax.axis_index("s")
        for c in range(NCH):
            pltpu.sync_copy(dest_hbm.at[wid * NCH + c], idx_vm)
            pltpu.sync_copy(outs_hbm.at[idx_vm], rows_vm)
            pltpu.sync_copy(rows_vm, o_hbm.at[pl.ds(wid * TOK_W + c * CHUNK,
                                                    CHUNK)])

    return k4(outs, dest)


def _metadata(counts, starts, ends):
    t_first = starts // TM
    t_last = jnp.maximum(t_first, (ends - 1) // TM)
    nsteps = jnp.where(counts > 0, t_last - t_first + 1, 0)
    total = jnp.sum(nsteps)
    sexp = jnp.repeat(jnp.arange(NEXP, dtype=jnp.int32), nsteps,
                      total_repeat_length=NT)
    base = jnp.cumsum(nsteps) - nsteps
    steps = jnp.arange(NT, dtype=jnp.int32)
    stile = t_first[sexp] + steps - base[sexp]
    valid = steps < total
    last = total - 1
    sexp = jnp.where(valid, sexp, jnp.take(sexp, last))
    stile = jnp.where(valid, stile, jnp.take(stile, last))
    stile = jnp.clip(stile, 0, N_TOK // TM - 1)
    sfirst = valid & (stile != jnp.roll(stile, 1))
    sfirst = sfirst.at[0].set(True)
    meta = jnp.stack([
        sexp,
        stile,
        sfirst.astype(jnp.int32),
        valid.astype(jnp.int32),
        starts[sexp],
        ends[sexp],
    ])
    return meta.astype(jnp.int32)


def kernel(x, Wr1, br1, Wr2, br2, We1, be1, We2, be2, Ws1, bs1, Ws2, bs2):
    B, T, C = x.shape
    x2 = x.reshape(N_TOK, C)

    logits = jax.nn.relu(x2 @ Wr1 + br1) @ Wr2 + br2
    gates = jax.nn.softmax(logits, axis=-1)
    _, topk_idx = jax.lax.top_k(gates, 1)
    e = topk_idx[:, 0].astype(jnp.int32)

    rank3, counts_f = _count_call(e.reshape(NCB, 1, CB))
    counts = counts_f[0].astype(jnp.int32)
    ends = jnp.cumsum(counts)
    starts = ends - counts

    xs, dest = _dispatch_call(x2, e, rank3.reshape(N_TOK), starts)

    meta = _metadata(counts, starts, ends)
    outs = _ffn_call(meta, xs, We1, be1, We2, be2,
                     Ws1, bs1.reshape(1, INTER), Ws2, bs2.reshape(1, HID))

    out = _combine_call(outs, dest)
    return out.reshape(B, T, C)
